# Initial kernel scaffold; baseline (speedup 1.0000x reference)
import jax
import jax.numpy as jnp
from jax.experimental import pallas as pl


def kernel(categorical_features, continous_features, emb_table, W, b):
    raise NotImplementedError("write your pallas kernel here")


# speedup vs baseline: 11.1153x; 11.1153x over previous
import functools

import jax
import jax.numpy as jnp
from jax import lax
from jax.experimental import pallas as pl
from jax.experimental.pallas import tpu as pltpu
from jax.experimental.pallas import tpu_sc as plsc

BATCH = 16384
CAT = 26
EMB = 32
CONT = 13
NC = 2
NS = 16
NW = NC * NS
BPW = BATCH // NW
QPW = BPW // 128
NCHUNK = CAT * QPW
CPAD = 16
WLEN = CAT * EMB + CPAD


def _body(cat_idx, cont_p, table, wf, out, idx_v, cont_v, rows_v, acc_v, w_v, sem):
    wid = lax.axis_index("s") * NC + lax.axis_index("c")
    base = wid * BPW
    pltpu.sync_copy(cat_idx.at[wid], idx_v)
    pltpu.sync_copy(cont_p.at[pl.ds(base, BPW)], cont_v)
    pltpu.sync_copy(wf, w_v)
    iota = lax.iota(jnp.int32, 16)

    wcb = [plsc.load_gather(w_v, [jnp.full((16,), CAT * EMB + d, jnp.int32)])
           for d in range(CONT + 1)]

    def cont_group(g, carry):
        ridx = g * 16 + iota
        a = jnp.zeros((16,), jnp.float32)
        for d in range(CONT + 1):
            v = plsc.load_gather(cont_v, [ridx, jnp.full((16,), d, jnp.int32)])
            a = a + v * wcb[d]
        acc_v[pl.ds(g * 16, 16)] = a
        return carry

    lax.fori_loop(0, BPW // 16, cont_group, 0)

    def chunk(c, carry):
        pltpu.async_copy(table.at[idx_v.at[c]], rows_v, sem).wait()
        woff = (c // QPW) * EMB
        wb = [plsc.load_gather(w_v, [jnp.full((16,), d, jnp.int32) + woff])
              for d in range(EMB)]
        qbase = (c % QPW) * 128

        def grp(g, inner_carry):
            ridx = g * 16 + iota
            ab = qbase + g * 16
            a = acc_v[pl.ds(ab, 16)]
            for d in range(EMB):
                v = plsc.load_gather(rows_v, [ridx, jnp.full((16,), d, jnp.int32)])
                a = a + v * wb[d]
            acc_v[pl.ds(ab, 16)] = a
            return inner_carry

        lax.fori_loop(0, 8, grp, 0)
        return carry

    lax.fori_loop(0, NCHUNK, chunk, 0)

    def sig(g, carry):
        x = acc_v[pl.ds(g * 16, 16)]
        acc_v[pl.ds(g * 16, 16)] = 1.0 / (1.0 + jnp.exp(-x))
        return carry

    lax.fori_loop(0, BPW // 16, sig, 0)
    pltpu.sync_copy(acc_v, out.at[pl.ds(base, BPW)])


@functools.partial(jax.jit)
def _run(cat_idx, cont_p, table, wf):
    mesh = plsc.VectorSubcoreMesh(core_axis_name="c", subcore_axis_name="s")
    f = pl.kernel(
        _body,
        mesh=mesh,
        compiler_params=pltpu.CompilerParams(
            needs_layout_passes=False, use_tc_tiling_on_sc=False),
        out_type=jax.ShapeDtypeStruct((BATCH,), jnp.float32),
        scratch_types=[
            pltpu.VMEM((NCHUNK, 128), jnp.int32),
            pltpu.VMEM((BPW, CPAD), jnp.float32),
            pltpu.VMEM((128, EMB), jnp.float32),
            pltpu.VMEM((BPW,), jnp.float32),
            pltpu.VMEM((WLEN,), jnp.float32),
            pltpu.SemaphoreType.DMA,
        ],
    )
    return f(cat_idx, cont_p, table, wf)


def kernel(categorical_features, continous_features, emb_table, W, b):
    cat_idx = categorical_features.astype(jnp.int32)
    cat_idx = cat_idx.reshape(NW, QPW, 128, CAT).transpose(0, 3, 1, 2)
    cat_idx = cat_idx.reshape(NW, NCHUNK, 128)
    cont_p = jnp.concatenate(
        [continous_features,
         jnp.ones((BATCH, 1), jnp.float32),
         jnp.zeros((BATCH, CPAD - CONT - 1), jnp.float32)], axis=1)
    wf = jnp.concatenate(
        [W[:, 0], b, jnp.zeros((CPAD - CONT - 1,), jnp.float32)])
    out = _run(cat_idx, cont_p, emb_table, wf)
    return out.reshape(BATCH, 1)


# speedup vs baseline: 12.2978x; 1.1064x over previous
import functools

import jax
import jax.numpy as jnp
from jax import lax
from jax.experimental import pallas as pl
from jax.experimental.pallas import tpu as pltpu
from jax.experimental.pallas import tpu_sc as plsc

BATCH = 16384
CAT = 26
EMB = 32
CONT = 13
NC = 2
NS = 16
NW = NC * NS
BPW = BATCH // NW
QPW = BPW // 128
NCHUNK = CAT * QPW
CPAD = 16
WLEN = CAT * EMB + CPAD
NBUF = 4


def _body(cat_idx, cont_p, table, wf, out, idx_v, cont_v, rows_v, acc_v, w_v, sem):
    wid = lax.axis_index("s") * NC + lax.axis_index("c")
    base = wid * BPW
    pltpu.sync_copy(cat_idx.at[wid], idx_v)
    pltpu.sync_copy(cont_p.at[pl.ds(base, BPW)], cont_v)
    pltpu.sync_copy(wf, w_v)
    iota = lax.iota(jnp.int32, 16)

    wcb = [plsc.load_gather(w_v, [jnp.full((16,), CAT * EMB + d, jnp.int32)])
           for d in range(CONT + 1)]

    def cont_group(g, carry):
        ridx = g * 16 + iota
        a = jnp.zeros((16,), jnp.float32)
        for d in range(CONT + 1):
            v = plsc.load_gather(cont_v, [ridx, jnp.full((16,), d, jnp.int32)])
            a = a + v * wcb[d]
        acc_v[pl.ds(g * 16, 16)] = a
        return carry

    lax.fori_loop(0, BPW // 16, cont_group, 0)

    def fire(c, buf):
        pltpu.async_copy(table.at[idx_v.at[c]], rows_v.at[buf], sem.at[buf])

    for c0 in range(NBUF - 1):
        fire(c0, c0)

    def chunk(c, carry):
        buf = lax.rem(c, NBUF)

        @pl.when(c + NBUF - 1 < NCHUNK)
        def _():
            fire(c + NBUF - 1, lax.rem(c + NBUF - 1, NBUF))

        pltpu.make_async_copy(
            table.at[idx_v.at[c]], rows_v.at[buf], sem.at[buf]).wait()
        woff = (c // QPW) * EMB
        wb = [plsc.load_gather(w_v, [jnp.full((16,), d, jnp.int32) + woff])
              for d in range(EMB)]
        qbase = (c % QPW) * 128
        bidx = jnp.full((16,), buf, jnp.int32)

        def grp(g, inner_carry):
            ridx = g * 16 + iota
            ab = qbase + g * 16
            a0 = acc_v[pl.ds(ab, 16)]
            a1 = jnp.zeros((16,), jnp.float32)
            a2 = jnp.zeros((16,), jnp.float32)
            a3 = jnp.zeros((16,), jnp.float32)
            for d in range(0, EMB, 4):
                dd = [jnp.full((16,), d + k, jnp.int32) for k in range(4)]
                a0 = a0 + plsc.load_gather(rows_v, [bidx, ridx, dd[0]]) * wb[d]
                a1 = a1 + plsc.load_gather(rows_v, [bidx, ridx, dd[1]]) * wb[d + 1]
                a2 = a2 + plsc.load_gather(rows_v, [bidx, ridx, dd[2]]) * wb[d + 2]
                a3 = a3 + plsc.load_gather(rows_v, [bidx, ridx, dd[3]]) * wb[d + 3]
            acc_v[pl.ds(ab, 16)] = (a0 + a1) + (a2 + a3)
            return inner_carry

        lax.fori_loop(0, 8, grp, 0)
        return carry

    lax.fori_loop(0, NCHUNK, chunk, 0)

    def sig(g, carry):
        x = acc_v[pl.ds(g * 16, 16)]
        acc_v[pl.ds(g * 16, 16)] = 1.0 / (1.0 + jnp.exp(-x))
        return carry

    lax.fori_loop(0, BPW // 16, sig, 0)
    pltpu.sync_copy(acc_v, out.at[pl.ds(base, BPW)])


@functools.partial(jax.jit)
def _run(cat_idx, cont_p, table, wf):
    mesh = plsc.VectorSubcoreMesh(core_axis_name="c", subcore_axis_name="s")
    f = pl.kernel(
        _body,
        mesh=mesh,
        compiler_params=pltpu.CompilerParams(
            needs_layout_passes=False, use_tc_tiling_on_sc=False),
        out_type=jax.ShapeDtypeStruct((BATCH,), jnp.float32),
        scratch_types=[
            pltpu.VMEM((NCHUNK, 128), jnp.int32),
            pltpu.VMEM((BPW, CPAD), jnp.float32),
            pltpu.VMEM((NBUF, 128, EMB), jnp.float32),
            pltpu.VMEM((BPW,), jnp.float32),
            pltpu.VMEM((WLEN,), jnp.float32),
            pltpu.SemaphoreType.DMA((NBUF,)),
        ],
    )
    return f(cat_idx, cont_p, table, wf)


def kernel(categorical_features, continous_features, emb_table, W, b):
    cat_idx = categorical_features.astype(jnp.int32)
    cat_idx = cat_idx.reshape(NW, QPW, 128, CAT).transpose(0, 3, 1, 2)
    cat_idx = cat_idx.reshape(NW, NCHUNK, 128)
    cont_p = jnp.concatenate(
        [continous_features,
         jnp.ones((BATCH, 1), jnp.float32),
         jnp.zeros((BATCH, CPAD - CONT - 1), jnp.float32)], axis=1)
    wf = jnp.concatenate(
        [W[:, 0], b, jnp.zeros((CPAD - CONT - 1,), jnp.float32)])
    out = _run(cat_idx, cont_p, emb_table, wf)
    return out.reshape(BATCH, 1)
